# baseline (speedup 1.0000x reference)
import jax
import jax.numpy as jnp
from jax.experimental import pallas as pl
from jax.experimental.pallas import tpu as pltpu

_B = 8
_N = 2048
_K = 64
_QDIM = 128
_MLP_HID = 256
_HID = 128
_OUT = 128
_NCLS = 92
_RBLK = 256
_NRB = _N // _RBLK
_HI = jax.lax.Precision.HIGHEST
f32 = jnp.float32


def _kth_largest(adj):
    i = jax.lax.bitcast_convert_type(adj, jnp.int32)
    k = i ^ ((i >> 31) & jnp.int32(0x7FFFFFFF))
    s = jnp.full((adj.shape[0], 1), -2147483648, jnp.int32)
    for b in range(31, -1, -1):
        cand = jnp.zeros_like(s) if b == 31 else s + jnp.int32(1 << b)
        cnt = jnp.sum((k >= cand).astype(f32), axis=1, keepdims=True)
        s = jnp.where(cnt >= float(_K), cand, s)
    inv = s ^ ((s >> 31) & jnp.int32(0x7FFFFFFF))
    return jax.lax.bitcast_convert_type(inv, f32)


def _dis_of(deg):
    return jnp.where(deg > 0, jax.lax.rsqrt(deg), 0.0)


def _fused(qe_ref, pb_ref, qen_ref, pbn_ref, w1q_ref, w1pb_ref, b1_ref,
           w2_ref, b2_ref, ne_ref, c1w_ref, c1b_ref, bns_ref, bnb_ref,
           c2w_ref, c2b_ref,
           out_ref, m_ref, degr_ref, degc_ref, x1_ref, x2_ref, acc_ref,
           abuf_ref):
    ph = pl.program_id(1)
    rb = pl.program_id(2)
    sl = pl.ds(rb * _RBLK, _RBLK)

    def _adj_block(qe, pb):
        qh = jnp.dot(qe, w1q_ref[...], preferred_element_type=f32)
        phh = jnp.dot(pb, w1pb_ref[...], preferred_element_type=f32)
        h = jnp.maximum(qh + phh + b1_ref[...], 0.0)
        return jax.lax.dot_general(h, w2_ref[...], (((1,), (1,)), ((), ())),
                                   preferred_element_type=f32) + b2_ref[...]

    @pl.when(ph == 0)
    def _():
        @pl.when(rb == 0)
        def _():
            abuf_ref[0] = _adj_block(qe_ref[...], pb_ref[0])

        @pl.when(rb < _NRB - 1)
        def _():
            abuf_ref[(rb + 1) % 2] = _adj_block(qen_ref[...], pbn_ref[0])

        adj = abuf_ref[rb % 2]
        thr = _kth_largest(adj)
        rows = rb * _RBLK + jax.lax.broadcasted_iota(jnp.int32, (_RBLK, _N), 0)
        cols = jax.lax.broadcasted_iota(jnp.int32, (_RBLK, _N), 1)
        m = jnp.where((adj >= thr) & (rows != cols), adj, 0.0)
        m_ref[sl, :] = m
        dcontrib = jnp.sum(m, axis=0)[None, :]

        @pl.when(rb == 0)
        def _():
            degr_ref[...] = dcontrib + 1.0

        @pl.when(rb != 0)
        def _():
            degr_ref[...] = degr_ref[...] + dcontrib

    @pl.when(ph == 1)
    def _():
        @pl.when(rb == 0)
        def _():
            degc_ref[...] = degr_ref[0].reshape(_N, 1)
            x1_ref[...] = jax.lax.dot_general(
                ne_ref[...], c1w_ref[...], (((1,), (1,)), ((), ())),
                preferred_element_type=f32)

        z = x1_ref[sl, :] * _dis_of(degc_ref[sl, :])
        contrib = jax.lax.dot_general(m_ref[sl, :], z, (((0,), (0,)), ((), ())),
                                      preferred_element_type=f32)

        @pl.when(rb == 0)
        def _():
            acc_ref[...] = contrib

        @pl.when(rb != 0)
        def _():
            acc_ref[...] = acc_ref[...] + contrib

        @pl.when(rb == _NRB - 1)
        def _():
            dis = _dis_of(degc_ref[...])
            out1 = acc_ref[...] * dis + x1_ref[...] * (dis * dis) + c1b_ref[...]
            y = jnp.maximum(out1 * bns_ref[...] + bnb_ref[...], 0.0)
            x2_ref[...] = jax.lax.dot_general(
                y, c2w_ref[...], (((1,), (1,)), ((), ())),
                preferred_element_type=f32)

    @pl.when(ph == 2)
    def _():
        z = x2_ref[sl, :] * _dis_of(degc_ref[sl, :])
        contrib = jax.lax.dot_general(m_ref[sl, :], z, (((0,), (0,)), ((), ())),
                                      preferred_element_type=f32)

        @pl.when(rb == 0)
        def _():
            acc_ref[...] = contrib

        @pl.when(rb != 0)
        def _():
            acc_ref[...] = acc_ref[...] + contrib

        @pl.when(rb == _NRB - 1)
        def _():
            dis = _dis_of(degc_ref[...])
            out_ref[0] = (acc_ref[...] * dis + x2_ref[...] * (dis * dis)
                          + c2b_ref[...])


def kernel(probs, bbox_coords, query_emb, node_emb, mlp_w1, mlp_b1, mlp_w2,
           mlp_b2, conv1_w, conv1_b, conv2_w, conv2_b, bn_gamma, bn_beta):
    pb = jnp.concatenate([probs, bbox_coords], axis=-1)
    pb = jnp.pad(pb, ((0, 0), (0, 0), (0, 128 - (_NCLS + 4))))
    w1q_t = mlp_w1[:, :_QDIM].T
    w1pb_t = jnp.pad(mlp_w1[:, _QDIM:], ((0, 0), (0, 128 - (_NCLS + 4)))).T
    full = lambda shape: pl.BlockSpec(shape, lambda b, ph, rb: (0,) * len(shape))
    out = pl.pallas_call(
        _fused,
        grid=(_B, 3, _NRB),
        in_specs=[
            pl.BlockSpec((_RBLK, _QDIM),
                         lambda b, ph, rb: (jnp.where(ph == 0, rb, 0), 0)),
            pl.BlockSpec((1, _RBLK, 128),
                         lambda b, ph, rb: (b, jnp.where(ph == 0, rb, 0), 0)),
            pl.BlockSpec((_RBLK, _QDIM),
                         lambda b, ph, rb: (jnp.where(
                             ph == 0, jnp.minimum(rb + 1, _NRB - 1), 0), 0)),
            pl.BlockSpec((1, _RBLK, 128),
                         lambda b, ph, rb: (b, jnp.where(
                             ph == 0, jnp.minimum(rb + 1, _NRB - 1), 0), 0)),
            full((_QDIM, _MLP_HID)),
            full((128, _MLP_HID)),
            full((1, _MLP_HID)),
            full((_N, _MLP_HID)),
            full((1, _N)),
            full((_N, _QDIM)),
            full((_HID, _QDIM)),
            full((1, _HID)),
            full((1, _HID)),
            full((1, _HID)),
            full((_OUT, _HID)),
            full((1, _OUT)),
        ],
        out_specs=pl.BlockSpec((1, _N, _OUT), lambda b, ph, rb: (b, 0, 0)),
        out_shape=jax.ShapeDtypeStruct((_B, _N, _OUT), f32),
        scratch_shapes=[
            pltpu.VMEM((_N, _N), f32),
            pltpu.VMEM((1, _N), f32),
            pltpu.VMEM((_N, 1), f32),
            pltpu.VMEM((_N, _QDIM), f32),
            pltpu.VMEM((_N, _HID), f32),
            pltpu.VMEM((_N, _HID), f32),
            pltpu.VMEM((2, _RBLK, _N), f32),
        ],
    )(query_emb, pb, query_emb, pb, w1q_t, w1pb_t,
      mlp_b1.reshape(1, _MLP_HID), mlp_w2,
      mlp_b2.reshape(1, _N), node_emb, conv1_w, conv1_b.reshape(1, _HID),
      (bn_gamma / jnp.sqrt(1.0 + 1e-5)).reshape(1, _HID),
      bn_beta.reshape(1, _HID), conv2_w, conv2_b.reshape(1, _OUT))
    return out[:, None, :, :]


# speedup vs baseline: 1.1113x; 1.1113x over previous
import jax
import jax.numpy as jnp
from jax.experimental import pallas as pl
from jax.experimental.pallas import tpu as pltpu

_B = 8
_N = 2048
_K = 64
_QDIM = 128
_MLP_HID = 256
_HID = 128
_OUT = 128
_NCLS = 92
_RBLK = 256
_NRB = _N // _RBLK
f32 = jnp.float32


def _kth_largest(adj):
    i = jax.lax.bitcast_convert_type(adj, jnp.int32)
    k = i ^ ((i >> 31) & jnp.int32(0x7FFFFFFF))
    s = jnp.full((adj.shape[0], 1), -2147483648, jnp.int32)
    for b in range(31, -1, -1):
        cand = jnp.zeros_like(s) if b == 31 else s + jnp.int32(1 << b)
        cnt = jnp.sum((k >= cand).astype(f32), axis=1, keepdims=True)
        s = jnp.where(cnt >= float(_K), cand, s)
    inv = s ^ ((s >> 31) & jnp.int32(0x7FFFFFFF))
    return jax.lax.bitcast_convert_type(inv, f32)


def _dis_of(deg):
    return jnp.where(deg > 0, jax.lax.rsqrt(deg), 0.0)


def _fused(qe_ref, pb_ref, w1q_ref, w1pb_ref, b1_ref, w2_ref, b2_ref, ne_ref,
           c1w_ref, c1b_ref, bns_ref, bnb_ref, c2w_ref, c2b_ref,
           out_ref, m_ref, degr_ref):
    s = pl.program_id(1)

    @pl.when(s < _NRB)
    def _():
        qh = jnp.dot(qe_ref[...], w1q_ref[...], preferred_element_type=f32)
        phh = jnp.dot(pb_ref[0], w1pb_ref[...], preferred_element_type=f32)
        h = jnp.maximum(qh + phh + b1_ref[...], 0.0)
        adj = jax.lax.dot_general(h, w2_ref[...], (((1,), (1,)), ((), ())),
                                  preferred_element_type=f32) + b2_ref[...]
        thr = _kth_largest(adj)
        rows = s * _RBLK + jax.lax.broadcasted_iota(jnp.int32, (_RBLK, _N), 0)
        cols = jax.lax.broadcasted_iota(jnp.int32, (_RBLK, _N), 1)
        m = jnp.where((adj >= thr) & (rows != cols), adj, 0.0)
        m_ref[pl.ds(s * _RBLK, _RBLK), :] = m
        dcontrib = jnp.sum(m, axis=0)[None, :]

        @pl.when(s == 0)
        def _():
            degr_ref[...] = dcontrib + 1.0

        @pl.when(s != 0)
        def _():
            degr_ref[...] = degr_ref[...] + dcontrib

    @pl.when(s == _NRB)
    def _():
        dis = _dis_of(degr_ref[0].reshape(_N, 1))
        dis2 = dis * dis
        mm = m_ref[...]
        x1 = jax.lax.dot_general(ne_ref[...], c1w_ref[...],
                                 (((1,), (1,)), ((), ())),
                                 preferred_element_type=f32)
        t1 = jax.lax.dot_general(mm, x1 * dis, (((0,), (0,)), ((), ())),
                                 preferred_element_type=f32)
        out1 = t1 * dis + x1 * dis2 + c1b_ref[...]
        y = jnp.maximum(out1 * bns_ref[...] + bnb_ref[...], 0.0)
        x2 = jax.lax.dot_general(y, c2w_ref[...], (((1,), (1,)), ((), ())),
                                 preferred_element_type=f32)
        t2 = jax.lax.dot_general(mm, x2 * dis, (((0,), (0,)), ((), ())),
                                 preferred_element_type=f32)
        out_ref[0] = t2 * dis + x2 * dis2 + c2b_ref[...]


def kernel(probs, bbox_coords, query_emb, node_emb, mlp_w1, mlp_b1, mlp_w2,
           mlp_b2, conv1_w, conv1_b, conv2_w, conv2_b, bn_gamma, bn_beta):
    pb = jnp.concatenate([probs, bbox_coords], axis=-1)
    pb = jnp.pad(pb, ((0, 0), (0, 0), (0, 128 - (_NCLS + 4))))
    w1q_t = mlp_w1[:, :_QDIM].T
    w1pb_t = jnp.pad(mlp_w1[:, _QDIM:], ((0, 0), (0, 128 - (_NCLS + 4)))).T
    full = lambda shape: pl.BlockSpec(shape, lambda b, s: (0,) * len(shape))
    out = pl.pallas_call(
        _fused,
        grid=(_B, _NRB + 1),
        in_specs=[
            pl.BlockSpec((_RBLK, _QDIM),
                         lambda b, s: (jnp.minimum(s, _NRB - 1), 0)),
            pl.BlockSpec((1, _RBLK, 128),
                         lambda b, s: (b, jnp.minimum(s, _NRB - 1), 0)),
            full((_QDIM, _MLP_HID)),
            full((128, _MLP_HID)),
            full((1, _MLP_HID)),
            full((_N, _MLP_HID)),
            full((1, _N)),
            full((_N, _QDIM)),
            full((_HID, _QDIM)),
            full((1, _HID)),
            full((1, _HID)),
            full((1, _HID)),
            full((_OUT, _HID)),
            full((1, _OUT)),
        ],
        out_specs=pl.BlockSpec((1, _N, _OUT), lambda b, s: (b, 0, 0)),
        out_shape=jax.ShapeDtypeStruct((_B, _N, _OUT), f32),
        scratch_shapes=[
            pltpu.VMEM((_N, _N), f32),
            pltpu.VMEM((1, _N), f32),
        ],
    )(query_emb, pb, w1q_t, w1pb_t, mlp_b1.reshape(1, _MLP_HID), mlp_w2,
      mlp_b2.reshape(1, _N), node_emb, conv1_w, conv1_b.reshape(1, _HID),
      (bn_gamma / jnp.sqrt(1.0 + 1e-5)).reshape(1, _HID),
      bn_beta.reshape(1, _HID), conv2_w, conv2_b.reshape(1, _OUT))
    return out[:, None, :, :]
